# baseline (speedup 1.0000x reference)
import functools

import jax
import jax.numpy as jnp
from jax import lax
from jax.experimental import pallas as pl
from jax.experimental.pallas import tpu as pltpu
from jax.experimental.pallas import tpu_sc as plsc

N = 10000
E = 160000
D = 256
H = 128
LAT = 64
OUT = 4
GROUPS = 8
CG = D // GROUPS
EPS = 1e-5

NS = 16
BK = 128
IC = 8
NC = 10
CH = IC * NC
EPT = E // NS
PPT = CH * BK - EPT
PADROWS = 128
ACC_ROWS = N + PADROWS
SR = 640


def _agg_body(src_hbm, dst_hbm, xlo_hbm, xhi_hbm, zero_hbm,
              alo_hbm, ahi_hbm,
              src_v, dst_v, rows0_v, rows1_v, acc, gsem, ssem):
    c = lax.axis_index("c")
    s = lax.axis_index("s")

    zb = jnp.minimum(s * SR, ACC_ROWS - SR)
    pltpu.sync_copy(zero_hbm, acc.at[pl.ds(zb, SR)])
    plsc.subcore_barrier()

    def edge_loop(x_hbm):
        def gather(j, buf):
            pltpu.async_copy(x_hbm.at[src_v.at[j]], buf, gsem)

        def scat(j, buf):
            pltpu.async_copy(buf, acc.at[dst_v.at[j]], ssem, add=True)

        def gather_wait(j, buf):
            pltpu.make_async_copy(x_hbm.at[src_v.at[j]], buf, gsem).wait()

        def scat_wait(j, buf):
            pltpu.make_async_copy(buf, acc.at[dst_v.at[j]], ssem).wait()

        def chunk(ci, carry):
            pltpu.sync_copy(src_hbm.at[s, pl.ds(ci * IC, IC)], src_v)
            pltpu.sync_copy(dst_hbm.at[s, pl.ds(ci * IC, IC)], dst_v)
            gather(0, rows0_v)
            gather(1, rows1_v)

            def body(i, carry2):
                j = 2 * i
                gather_wait(j, rows0_v)
                scat(j, rows0_v)
                gather_wait(j + 1, rows1_v)
                scat(j + 1, rows1_v)
                scat_wait(j, rows0_v)

                @pl.when(j + 2 < IC)
                def _():
                    gather(j + 2, rows0_v)
                scat_wait(j + 1, rows1_v)

                @pl.when(j + 3 < IC)
                def _():
                    gather(j + 3, rows1_v)
                return carry2
            lax.fori_loop(0, IC // 2, body, 0)
            return carry
        lax.fori_loop(0, NC, chunk, 0)

    @pl.when(c == 0)
    def _():
        edge_loop(xlo_hbm)

    @pl.when(c == 1)
    def _():
        edge_loop(xhi_hbm)

    plsc.subcore_barrier()

    ob = jnp.minimum(s * SR, N - SR)

    @pl.when(c == 0)
    def _():
        pltpu.sync_copy(acc.at[pl.ds(ob, SR)], alo_hbm.at[pl.ds(ob, SR)])

    @pl.when(c == 1)
    def _():
        pltpu.sync_copy(acc.at[pl.ds(ob, SR)], ahi_hbm.at[pl.ds(ob, SR)])


@functools.cache
def _agg_kernel():
    return functools.partial(
        pl.kernel,
        out_type=(jax.ShapeDtypeStruct((N, H), jnp.float32),
                  jax.ShapeDtypeStruct((N, H), jnp.float32)),
        mesh=plsc.VectorSubcoreMesh(core_axis_name="c", subcore_axis_name="s"),
        scratch_types=[
            pltpu.VMEM((IC, BK), jnp.int32),
            pltpu.VMEM((IC, BK), jnp.int32),
            pltpu.VMEM((BK, H), jnp.float32),
            pltpu.VMEM((BK, H), jnp.float32),
            pltpu.VMEM_SHARED((ACC_ROWS, H), jnp.float32),
            pltpu.SemaphoreType.DMA,
            pltpu.SemaphoreType.DMA,
        ],
    )(_agg_body)


def _agg(src, dst, xlo, xhi, zero_rows):
    return _agg_kernel()(src, dst, xlo, xhi, zero_rows)


def _gn_gelu(h, gamma, beta):
    f32 = jnp.float32
    G = (lax.broadcasted_iota(jnp.int32, (D, GROUPS), 0) // CG
         == lax.broadcasted_iota(jnp.int32, (D, GROUPS), 1)).astype(f32)
    GT = (lax.broadcasted_iota(jnp.int32, (GROUPS, D), 0)
          == lax.broadcasted_iota(jnp.int32, (GROUPS, D), 1) // CG).astype(f32)
    s1 = jnp.dot(h, G, preferred_element_type=f32)
    s2 = jnp.dot(h * h, G, preferred_element_type=f32)
    mean = s1 * (1.0 / CG)
    var = s2 * (1.0 / CG) - mean * mean
    rstd = lax.rsqrt(var + EPS)
    rstdf = jnp.dot(rstd, GT, preferred_element_type=f32)
    mrf = jnp.dot(mean * rstd, GT, preferred_element_type=f32)
    y = (h * rstdf - mrf) * gamma + beta
    return jax.nn.gelu(y)


def _stage_a_body(x_ref, alo_ref, ahi_ref, nz_ref,
                  w1s_ref, wnlo_ref, wnhi_ref, b1_ref, g1_ref, be1_ref,
                  wmm_ref, wmv_ref, bmm_ref, bmv_ref, wup_ref, bup_ref,
                  dlo_ref, dhi_ref):
    f32 = jnp.float32
    h = (jnp.dot(x_ref[...], w1s_ref[...], preferred_element_type=f32)
         + jnp.dot(alo_ref[...], wnlo_ref[...], preferred_element_type=f32)
         + jnp.dot(ahi_ref[...], wnhi_ref[...], preferred_element_type=f32)
         + b1_ref[...])
    h = _gn_gelu(h, g1_ref[...], be1_ref[...])
    mz = jnp.dot(h, wmm_ref[...], preferred_element_type=f32) + bmm_ref[...]
    lv = jnp.dot(h, wmv_ref[...], preferred_element_type=f32) + bmv_ref[...]
    lv = jnp.clip(lv, -30.0, 20.0)
    z = mz + jnp.exp(0.5 * lv) * nz_ref[...]
    d0 = jax.nn.gelu(jnp.dot(z, wup_ref[...], preferred_element_type=f32)
                     + bup_ref[...])
    dlo_ref[...] = d0[:, :H]
    dhi_ref[...] = d0[:, H:]


def _stage_b_body(dlo_ref, dhi_ref, alo_ref, ahi_ref,
                  wslo_ref, wshi_ref, wnlo_ref, wnhi_ref,
                  b2_ref, g2_ref, be2_ref, wout_ref, bout_ref,
                  out_ref):
    f32 = jnp.float32
    d = (jnp.dot(dlo_ref[...], wslo_ref[...], preferred_element_type=f32)
         + jnp.dot(dhi_ref[...], wshi_ref[...], preferred_element_type=f32)
         + jnp.dot(alo_ref[...], wnlo_ref[...], preferred_element_type=f32)
         + jnp.dot(ahi_ref[...], wnhi_ref[...], preferred_element_type=f32)
         + b2_ref[...])
    d = _gn_gelu(d, g2_ref[...], be2_ref[...])
    out_ref[...] = (jnp.dot(d, wout_ref[...], preferred_element_type=f32)
                    + bout_ref[...])


_BR = 1000


def _row_spec(w):
    return pl.BlockSpec((_BR, w), lambda i: (i, 0))


def _full_spec(shape):
    return pl.BlockSpec(shape, lambda i: tuple(0 for _ in shape))


def _stage_a(x, alo, ahi, nz, w1s, wnlo, wnhi, b1, g1, be1,
             wmm, wmv, bmm, bmv, wup, bup):
    full = [_full_spec(a.shape) for a in
            (w1s, wnlo, wnhi, b1, g1, be1, wmm, wmv, bmm, bmv, wup, bup)]
    return pl.pallas_call(
        _stage_a_body,
        grid=(N // _BR,),
        in_specs=[_row_spec(D), _row_spec(H), _row_spec(H), _row_spec(LAT)] + full,
        out_specs=(_row_spec(H), _row_spec(H)),
        out_shape=(jax.ShapeDtypeStruct((N, H), jnp.float32),
                   jax.ShapeDtypeStruct((N, H), jnp.float32)),
    )(x, alo, ahi, nz, w1s, wnlo, wnhi, b1, g1, be1,
      wmm, wmv, bmm, bmv, wup, bup)


def _stage_b(dlo, dhi, alo, ahi, wslo, wshi, wnlo, wnhi, b2, g2, be2,
             wout, bout):
    full = [_full_spec(a.shape) for a in
            (wslo, wshi, wnlo, wnhi, b2, g2, be2, wout, bout)]
    return pl.pallas_call(
        _stage_b_body,
        grid=(N // _BR,),
        in_specs=[_row_spec(H), _row_spec(H), _row_spec(H), _row_spec(H)] + full,
        out_specs=_row_spec(OUT),
        out_shape=jax.ShapeDtypeStruct((N, OUT), jnp.float32),
    )(dlo, dhi, alo, ahi, wslo, wshi, wnlo, wnhi, b2, g2, be2, wout, bout)


def kernel(x, edge_index, W1s, W1n, b1, g1, be1, Wmu, bmu, Wup, bup,
           W2s, W2n, b2, g2, be2, Wout, bout, noise):
    ei = edge_index.astype(jnp.int32)
    pad_src = jnp.zeros((NS, PPT), jnp.int32)
    pad_dst = jnp.broadcast_to(
        N + (jnp.arange(PPT, dtype=jnp.int32) % PADROWS), (NS, PPT))
    src = jnp.concatenate([ei[0].reshape(NS, EPT), pad_src], axis=1)
    src = src.reshape(NS, CH, BK)
    dst = jnp.concatenate([ei[1].reshape(NS, EPT), pad_dst], axis=1)
    dst = dst.reshape(NS, CH, BK)
    zero_rows = jnp.zeros((SR, H), jnp.float32)

    x_lo = x[:, :H]
    x_hi = x[:, H:]
    a1_lo, a1_hi = _agg(src, dst, x_lo, x_hi, zero_rows)

    d0_lo, d0_hi = _stage_a(
        x, a1_lo, a1_hi, noise,
        W1s, W1n[:H], W1n[H:],
        b1.reshape(1, D), g1.reshape(1, D), be1.reshape(1, D),
        Wmu[:, :LAT], Wmu[:, LAT:],
        bmu[:LAT].reshape(1, LAT), bmu[LAT:].reshape(1, LAT),
        Wup, bup.reshape(1, D))

    a2_lo, a2_hi = _agg(src, dst, d0_lo, d0_hi, zero_rows)

    return _stage_b(
        d0_lo, d0_hi, a2_lo, a2_hi,
        W2s[:H], W2s[H:], W2n[:H], W2n[H:],
        b2.reshape(1, D), g2.reshape(1, D), be2.reshape(1, D),
        Wout, bout.reshape(1, OUT))


# speedup vs baseline: 1.0749x; 1.0749x over previous
import functools

import jax
import jax.numpy as jnp
from jax import lax
from jax.experimental import pallas as pl
from jax.experimental.pallas import tpu as pltpu
from jax.experimental.pallas import tpu_sc as plsc

N = 10000
E = 160000
D = 256
H = 128
LAT = 64
OUT = 4
GROUPS = 8
CG = D // GROUPS
EPS = 1e-5

NS = 16
BK = 128
CH = 79
EPT = E // NS
PPT = CH * BK - EPT
PADROWS = 128
ACC_ROWS = N + PADROWS
SR = 640


def _agg_body(srclo_hbm, srchi_hbm, dst_hbm, x2_hbm, zero_hbm,
              alo_hbm, ahi_hbm,
              src_v, dst_v, rows_v, acc, sem):
    c = lax.axis_index("c")
    s = lax.axis_index("s")

    zb = jnp.minimum(s * SR, ACC_ROWS - SR)
    pltpu.sync_copy(zero_hbm, acc.at[pl.ds(zb, SR)])
    @pl.when(c == 0)
    def _():
        pltpu.sync_copy(srclo_hbm.at[s], src_v)

    @pl.when(c == 1)
    def _():
        pltpu.sync_copy(srchi_hbm.at[s], src_v)
    pltpu.sync_copy(dst_hbm.at[s], dst_v)
    plsc.subcore_barrier()

    def body(j, carry):
        pltpu.async_copy(x2_hbm.at[src_v.at[j]], rows_v, sem).wait()
        pltpu.sync_copy(rows_v, acc.at[dst_v.at[j]], add=True)
        return carry
    lax.fori_loop(0, CH, body, 0)

    plsc.subcore_barrier()

    ob = jnp.minimum(s * SR, N - SR)

    @pl.when(c == 0)
    def _():
        pltpu.sync_copy(acc.at[pl.ds(ob, SR)], alo_hbm.at[pl.ds(ob, SR)])

    @pl.when(c == 1)
    def _():
        pltpu.sync_copy(acc.at[pl.ds(ob, SR)], ahi_hbm.at[pl.ds(ob, SR)])


@functools.cache
def _agg_kernel():
    return functools.partial(
        pl.kernel,
        out_type=(jax.ShapeDtypeStruct((N, H), jnp.float32),
                  jax.ShapeDtypeStruct((N, H), jnp.float32)),
        mesh=plsc.VectorSubcoreMesh(core_axis_name="c", subcore_axis_name="s"),
        scratch_types=[
            pltpu.VMEM((CH, BK), jnp.int32),
            pltpu.VMEM((CH, BK), jnp.int32),
            pltpu.VMEM((BK, H), jnp.float32),
            pltpu.VMEM_SHARED((ACC_ROWS, H), jnp.float32),
            pltpu.SemaphoreType.DMA,
        ],
    )(_agg_body)


def _agg(src, dst, xlo, xhi, zero_rows):
    return _agg_kernel()(src, dst, xlo, xhi, zero_rows)


def _gn_gelu(h, gamma, beta):
    f32 = jnp.float32
    G = (lax.broadcasted_iota(jnp.int32, (D, GROUPS), 0) // CG
         == lax.broadcasted_iota(jnp.int32, (D, GROUPS), 1)).astype(f32)
    GT = (lax.broadcasted_iota(jnp.int32, (GROUPS, D), 0)
          == lax.broadcasted_iota(jnp.int32, (GROUPS, D), 1) // CG).astype(f32)
    s1 = jnp.dot(h, G, preferred_element_type=f32)
    s2 = jnp.dot(h * h, G, preferred_element_type=f32)
    mean = s1 * (1.0 / CG)
    var = s2 * (1.0 / CG) - mean * mean
    rstd = lax.rsqrt(var + EPS)
    rstdf = jnp.dot(rstd, GT, preferred_element_type=f32)
    mrf = jnp.dot(mean * rstd, GT, preferred_element_type=f32)
    y = (h * rstdf - mrf) * gamma + beta
    return jax.nn.gelu(y)


def _stage_a_body(x_ref, alo_ref, ahi_ref, nz_ref,
                  w1s_ref, wnlo_ref, wnhi_ref, b1_ref, g1_ref, be1_ref,
                  wmm_ref, wmv_ref, bmm_ref, bmv_ref, wup_ref, bup_ref,
                  d0_ref):
    f32 = jnp.float32
    h = (jnp.dot(x_ref[...], w1s_ref[...], preferred_element_type=f32)
         + jnp.dot(alo_ref[...], wnlo_ref[...], preferred_element_type=f32)
         + jnp.dot(ahi_ref[...], wnhi_ref[...], preferred_element_type=f32)
         + b1_ref[...])
    h = _gn_gelu(h, g1_ref[...], be1_ref[...])
    mz = jnp.dot(h, wmm_ref[...], preferred_element_type=f32) + bmm_ref[...]
    lv = jnp.dot(h, wmv_ref[...], preferred_element_type=f32) + bmv_ref[...]
    lv = jnp.clip(lv, -30.0, 20.0)
    z = mz + jnp.exp(0.5 * lv) * nz_ref[...]
    d0_ref[...] = jax.nn.gelu(jnp.dot(z, wup_ref[...], preferred_element_type=f32)
                              + bup_ref[...])


def _stage_b_body(d0_ref, alo_ref, ahi_ref,
                  ws_ref, wnlo_ref, wnhi_ref,
                  b2_ref, g2_ref, be2_ref, wout_ref, bout_ref,
                  out_ref):
    f32 = jnp.float32
    d = (jnp.dot(d0_ref[...], ws_ref[...], preferred_element_type=f32)
         + jnp.dot(alo_ref[...], wnlo_ref[...], preferred_element_type=f32)
         + jnp.dot(ahi_ref[...], wnhi_ref[...], preferred_element_type=f32)
         + b2_ref[...])
    d = _gn_gelu(d, g2_ref[...], be2_ref[...])
    out_ref[...] = (jnp.dot(d, wout_ref[...], preferred_element_type=f32)
                    + bout_ref[...])


_BR = 1000


def _row_spec(w):
    return pl.BlockSpec((_BR, w), lambda i: (i, 0))


def _full_spec(shape):
    return pl.BlockSpec(shape, lambda i: tuple(0 for _ in shape))


def _stage_a(x, alo, ahi, nz, w1s, wnlo, wnhi, b1, g1, be1,
             wmm, wmv, bmm, bmv, wup, bup):
    full = [_full_spec(a.shape) for a in
            (w1s, wnlo, wnhi, b1, g1, be1, wmm, wmv, bmm, bmv, wup, bup)]
    return pl.pallas_call(
        _stage_a_body,
        grid=(N // _BR,),
        in_specs=[_row_spec(D), _row_spec(H), _row_spec(H), _row_spec(LAT)] + full,
        out_specs=_row_spec(D),
        out_shape=jax.ShapeDtypeStruct((N, D), jnp.float32),
    )(x, alo, ahi, nz, w1s, wnlo, wnhi, b1, g1, be1,
      wmm, wmv, bmm, bmv, wup, bup)


def _stage_b(d0, alo, ahi, ws, wnlo, wnhi, b2, g2, be2, wout, bout):
    full = [_full_spec(a.shape) for a in
            (ws, wnlo, wnhi, b2, g2, be2, wout, bout)]
    return pl.pallas_call(
        _stage_b_body,
        grid=(N // _BR,),
        in_specs=[_row_spec(D), _row_spec(H), _row_spec(H)] + full,
        out_specs=_row_spec(OUT),
        out_shape=jax.ShapeDtypeStruct((N, OUT), jnp.float32),
    )(d0, alo, ahi, ws, wnlo, wnhi, b2, g2, be2, wout, bout)


def kernel(x, edge_index, W1s, W1n, b1, g1, be1, Wmu, bmu, Wup, bup,
           W2s, W2n, b2, g2, be2, Wout, bout, noise):
    ei = edge_index.astype(jnp.int32)
    pad_src = jnp.zeros((NS, PPT), jnp.int32)
    pad_dst = jnp.broadcast_to(
        N + (jnp.arange(PPT, dtype=jnp.int32) % PADROWS), (NS, PPT))
    src = jnp.concatenate([ei[0].reshape(NS, EPT), pad_src], axis=1)
    src = src.reshape(NS, CH, BK)
    dst = jnp.concatenate([ei[1].reshape(NS, EPT), pad_dst], axis=1)
    dst = dst.reshape(NS, CH, BK)
    srclo = src * 2
    srchi = srclo + 1
    zero_rows = jnp.zeros((SR, H), jnp.float32)

    a1_lo, a1_hi = _agg(srclo, srchi, dst, x.reshape(2 * N, H), zero_rows)

    d0 = _stage_a(
        x, a1_lo, a1_hi, noise,
        W1s, W1n[:H], W1n[H:],
        b1.reshape(1, D), g1.reshape(1, D), be1.reshape(1, D),
        Wmu[:, :LAT], Wmu[:, LAT:],
        bmu[:LAT].reshape(1, LAT), bmu[LAT:].reshape(1, LAT),
        Wup, bup.reshape(1, D))

    a2_lo, a2_hi = _agg(srclo, srchi, dst, d0.reshape(2 * N, H), zero_rows)

    return _stage_b(
        d0, a2_lo, a2_hi,
        W2s, W2n[:H], W2n[H:],
        b2.reshape(1, D), g2.reshape(1, D), be2.reshape(1, D),
        Wout, bout.reshape(1, OUT))


# speedup vs baseline: 1.1004x; 1.0237x over previous
import functools

import jax
import jax.numpy as jnp
from jax import lax
from jax.experimental import pallas as pl
from jax.experimental.pallas import tpu as pltpu
from jax.experimental.pallas import tpu_sc as plsc

N = 10000
E = 160000
D = 256
H = 128
LAT = 64
OUT = 4
GROUPS = 8
CG = D // GROUPS
EPS = 1e-5

NS = 16
BK = 128
CH = 79
EPT = E // NS
PPT = CH * BK - EPT
PADROWS = 128
ACC_ROWS = N + PADROWS
SR = 640


def _agg_body(src_hbm, dst_hbm, xlo_hbm, xhi_hbm, zero_hbm,
              alo_hbm, ahi_hbm,
              src_v, dst_v, rows_v, acc, sem):
    c = lax.axis_index("c")
    s = lax.axis_index("s")

    zb = jnp.minimum(s * SR, ACC_ROWS - SR)
    pltpu.sync_copy(zero_hbm, acc.at[pl.ds(zb, SR)])
    pltpu.sync_copy(src_hbm.at[s], src_v)
    pltpu.sync_copy(dst_hbm.at[s], dst_v)
    plsc.subcore_barrier()

    def edge_loop(x_hbm):
        def body(j, carry):
            pltpu.async_copy(x_hbm.at[src_v.at[j]], rows_v, sem).wait()
            pltpu.sync_copy(rows_v, acc.at[dst_v.at[j]], add=True)
            return carry
        lax.fori_loop(0, CH, body, 0)

    @pl.when(c == 0)
    def _():
        edge_loop(xlo_hbm)

    @pl.when(c == 1)
    def _():
        edge_loop(xhi_hbm)

    plsc.subcore_barrier()

    ob = jnp.minimum(s * SR, N - SR)

    @pl.when(c == 0)
    def _():
        pltpu.sync_copy(acc.at[pl.ds(ob, SR)], alo_hbm.at[pl.ds(ob, SR)])

    @pl.when(c == 1)
    def _():
        pltpu.sync_copy(acc.at[pl.ds(ob, SR)], ahi_hbm.at[pl.ds(ob, SR)])


@functools.cache
def _agg_kernel():
    return functools.partial(
        pl.kernel,
        out_type=(jax.ShapeDtypeStruct((N, H), jnp.float32),
                  jax.ShapeDtypeStruct((N, H), jnp.float32)),
        mesh=plsc.VectorSubcoreMesh(core_axis_name="c", subcore_axis_name="s"),
        scratch_types=[
            pltpu.VMEM((CH, BK), jnp.int32),
            pltpu.VMEM((CH, BK), jnp.int32),
            pltpu.VMEM((BK, H), jnp.float32),
            pltpu.VMEM_SHARED((ACC_ROWS, H), jnp.float32),
            pltpu.SemaphoreType.DMA,
        ],
    )(_agg_body)


def _agg(src, dst, xlo, xhi, zero_rows):
    return _agg_kernel()(src, dst, xlo, xhi, zero_rows)


def _gn_gelu(h, gamma, beta):
    f32 = jnp.float32
    G = (lax.broadcasted_iota(jnp.int32, (D, GROUPS), 0) // CG
         == lax.broadcasted_iota(jnp.int32, (D, GROUPS), 1)).astype(f32)
    GT = (lax.broadcasted_iota(jnp.int32, (GROUPS, D), 0)
          == lax.broadcasted_iota(jnp.int32, (GROUPS, D), 1) // CG).astype(f32)
    s1 = jnp.dot(h, G, preferred_element_type=f32)
    s2 = jnp.dot(h * h, G, preferred_element_type=f32)
    mean = s1 * (1.0 / CG)
    var = s2 * (1.0 / CG) - mean * mean
    rstd = lax.rsqrt(var + EPS)
    rstdf = jnp.dot(rstd, GT, preferred_element_type=f32)
    mrf = jnp.dot(mean * rstd, GT, preferred_element_type=f32)
    y = (h * rstdf - mrf) * gamma + beta
    return jax.nn.gelu(y)


def _stage_a_body(x_ref, alo_ref, ahi_ref, nz_ref,
                  w1s_ref, wnlo_ref, wnhi_ref, b1_ref, g1_ref, be1_ref,
                  wmm_ref, wmv_ref, bmm_ref, bmv_ref, wup_ref, bup_ref,
                  dlo_ref, dhi_ref):
    f32 = jnp.float32
    h = (jnp.dot(x_ref[...], w1s_ref[...], preferred_element_type=f32)
         + jnp.dot(alo_ref[...], wnlo_ref[...], preferred_element_type=f32)
         + jnp.dot(ahi_ref[...], wnhi_ref[...], preferred_element_type=f32)
         + b1_ref[...])
    h = _gn_gelu(h, g1_ref[...], be1_ref[...])
    mz = jnp.dot(h, wmm_ref[...], preferred_element_type=f32) + bmm_ref[...]
    lv = jnp.dot(h, wmv_ref[...], preferred_element_type=f32) + bmv_ref[...]
    lv = jnp.clip(lv, -30.0, 20.0)
    z = mz + jnp.exp(0.5 * lv) * nz_ref[...]
    d0 = jax.nn.gelu(jnp.dot(z, wup_ref[...], preferred_element_type=f32)
                     + bup_ref[...])
    dlo_ref[...] = d0[:, :H]
    dhi_ref[...] = d0[:, H:]


def _stage_b_body(dlo_ref, dhi_ref, alo_ref, ahi_ref,
                  wslo_ref, wshi_ref, wnlo_ref, wnhi_ref,
                  b2_ref, g2_ref, be2_ref, wout_ref, bout_ref,
                  out_ref):
    f32 = jnp.float32
    d = (jnp.dot(dlo_ref[...], wslo_ref[...], preferred_element_type=f32)
         + jnp.dot(dhi_ref[...], wshi_ref[...], preferred_element_type=f32)
         + jnp.dot(alo_ref[...], wnlo_ref[...], preferred_element_type=f32)
         + jnp.dot(ahi_ref[...], wnhi_ref[...], preferred_element_type=f32)
         + b2_ref[...])
    d = _gn_gelu(d, g2_ref[...], be2_ref[...])
    out_ref[...] = (jnp.dot(d, wout_ref[...], preferred_element_type=f32)
                    + bout_ref[...])


_BR = 1000


def _row_spec(w):
    return pl.BlockSpec((_BR, w), lambda i: (i, 0))


def _full_spec(shape):
    return pl.BlockSpec(shape, lambda i: tuple(0 for _ in shape))


def _stage_a(x, alo, ahi, nz, w1s, wnlo, wnhi, b1, g1, be1,
             wmm, wmv, bmm, bmv, wup, bup):
    full = [_full_spec(a.shape) for a in
            (w1s, wnlo, wnhi, b1, g1, be1, wmm, wmv, bmm, bmv, wup, bup)]
    return pl.pallas_call(
        _stage_a_body,
        grid=(N // _BR,),
        in_specs=[_row_spec(D), _row_spec(H), _row_spec(H), _row_spec(LAT)] + full,
        out_specs=(_row_spec(H), _row_spec(H)),
        out_shape=(jax.ShapeDtypeStruct((N, H), jnp.float32),
                   jax.ShapeDtypeStruct((N, H), jnp.float32)),
    )(x, alo, ahi, nz, w1s, wnlo, wnhi, b1, g1, be1,
      wmm, wmv, bmm, bmv, wup, bup)


def _stage_b(dlo, dhi, alo, ahi, wslo, wshi, wnlo, wnhi, b2, g2, be2,
             wout, bout):
    full = [_full_spec(a.shape) for a in
            (wslo, wshi, wnlo, wnhi, b2, g2, be2, wout, bout)]
    return pl.pallas_call(
        _stage_b_body,
        grid=(N // _BR,),
        in_specs=[_row_spec(H), _row_spec(H), _row_spec(H), _row_spec(H)] + full,
        out_specs=_row_spec(OUT),
        out_shape=jax.ShapeDtypeStruct((N, OUT), jnp.float32),
    )(dlo, dhi, alo, ahi, wslo, wshi, wnlo, wnhi, b2, g2, be2, wout, bout)


def kernel(x, edge_index, W1s, W1n, b1, g1, be1, Wmu, bmu, Wup, bup,
           W2s, W2n, b2, g2, be2, Wout, bout, noise):
    ei = edge_index.astype(jnp.int32)
    pad_src = jnp.zeros((NS, PPT), jnp.int32)
    pad_dst = jnp.broadcast_to(
        N + (jnp.arange(PPT, dtype=jnp.int32) % PADROWS), (NS, PPT))
    src = jnp.concatenate([ei[0].reshape(NS, EPT), pad_src], axis=1)
    src = src.reshape(NS, CH, BK)
    dst = jnp.concatenate([ei[1].reshape(NS, EPT), pad_dst], axis=1)
    dst = dst.reshape(NS, CH, BK)
    zero_rows = jnp.zeros((SR, H), jnp.float32)

    a1_lo, a1_hi = _agg(src, dst, x[:, :H], x[:, H:], zero_rows)

    d0_lo, d0_hi = _stage_a(
        x, a1_lo, a1_hi, noise,
        W1s, W1n[:H], W1n[H:],
        b1.reshape(1, D), g1.reshape(1, D), be1.reshape(1, D),
        Wmu[:, :LAT], Wmu[:, LAT:],
        bmu[:LAT].reshape(1, LAT), bmu[LAT:].reshape(1, LAT),
        Wup, bup.reshape(1, D))

    a2_lo, a2_hi = _agg(src, dst, d0_lo, d0_hi, zero_rows)

    return _stage_b(
        d0_lo, d0_hi, a2_lo, a2_hi,
        W2s[:H], W2s[H:], W2n[:H], W2n[H:],
        b2.reshape(1, D), g2.reshape(1, D), be2.reshape(1, D),
        Wout, bout.reshape(1, OUT))


# speedup vs baseline: 1.1053x; 1.0045x over previous
import functools

import jax
import jax.numpy as jnp
from jax import lax
from jax.experimental import pallas as pl
from jax.experimental.pallas import tpu as pltpu
from jax.experimental.pallas import tpu_sc as plsc

N = 10000
E = 160000
D = 256
H = 128
LAT = 64
OUT = 4
GROUPS = 8
CG = D // GROUPS
EPS = 1e-5

NS = 16
BK = 128
CH = 79
EPT = E // NS
PPT = CH * BK - EPT
PADROWS = 128
ACC_ROWS = N + PADROWS
SR = 640


def _agg_body(src_hbm, dst_hbm, xlo_hbm, xhi_hbm, zero_hbm,
              alo_hbm, ahi_hbm,
              src_v, dst_v, rows_v, acc, sem):
    c = lax.axis_index("c")
    s = lax.axis_index("s")

    zb = jnp.minimum(s * SR, ACC_ROWS - SR)
    pltpu.sync_copy(zero_hbm, acc.at[pl.ds(zb, SR)])
    pltpu.sync_copy(src_hbm.at[s], src_v)
    pltpu.sync_copy(dst_hbm.at[s], dst_v)
    plsc.subcore_barrier()

    def edge_loop(x_hbm):
        def body(j, carry):
            pltpu.async_copy(x_hbm.at[src_v.at[j]], rows_v, sem).wait()
            pltpu.sync_copy(rows_v, acc.at[dst_v.at[j]], add=True)
            return carry
        lax.fori_loop(0, CH, body, 0)

    @pl.when(c == 0)
    def _():
        edge_loop(xlo_hbm)

    @pl.when(c == 1)
    def _():
        edge_loop(xhi_hbm)

    plsc.subcore_barrier()

    ob = jnp.minimum(s * SR, N - SR)

    @pl.when(c == 0)
    def _():
        pltpu.sync_copy(acc.at[pl.ds(ob, SR)], alo_hbm.at[pl.ds(ob, SR)])

    @pl.when(c == 1)
    def _():
        pltpu.sync_copy(acc.at[pl.ds(ob, SR)], ahi_hbm.at[pl.ds(ob, SR)])


@functools.cache
def _agg_kernel():
    return functools.partial(
        pl.kernel,
        out_type=(jax.ShapeDtypeStruct((N, H), jnp.float32),
                  jax.ShapeDtypeStruct((N, H), jnp.float32)),
        mesh=plsc.VectorSubcoreMesh(core_axis_name="c", subcore_axis_name="s"),
        scratch_types=[
            pltpu.VMEM((CH, BK), jnp.int32),
            pltpu.VMEM((CH, BK), jnp.int32),
            pltpu.VMEM((BK, H), jnp.float32),
            pltpu.VMEM_SHARED((ACC_ROWS, H), jnp.float32),
            pltpu.SemaphoreType.DMA,
        ],
    )(_agg_body)


def _agg(src, dst, xlo, xhi, zero_rows):
    return _agg_kernel()(src, dst, xlo, xhi, zero_rows)


def _gn_gelu(h, gamma, beta):
    f32 = jnp.float32
    G = (lax.broadcasted_iota(jnp.int32, (D, GROUPS), 0) // CG
         == lax.broadcasted_iota(jnp.int32, (D, GROUPS), 1)).astype(f32)
    GT = (lax.broadcasted_iota(jnp.int32, (GROUPS, D), 0)
          == lax.broadcasted_iota(jnp.int32, (GROUPS, D), 1) // CG).astype(f32)
    s1 = jnp.dot(h, G, preferred_element_type=f32)
    s2 = jnp.dot(h * h, G, preferred_element_type=f32)
    mean = s1 * (1.0 / CG)
    var = s2 * (1.0 / CG) - mean * mean
    rstd = lax.rsqrt(var + EPS)
    rstdf = jnp.dot(rstd, GT, preferred_element_type=f32)
    mrf = jnp.dot(mean * rstd, GT, preferred_element_type=f32)
    y = (h * rstdf - mrf) * gamma + beta
    return jax.nn.gelu(y)


def _stage_a_body(x_ref, alo_ref, ahi_ref, nz_ref,
                  w1s_ref, wnlo_ref, wnhi_ref, b1_ref, g1_ref, be1_ref,
                  wmm_ref, wmv_ref, bmm_ref, bmv_ref, wup_ref, bup_ref,
                  dlo_ref, dhi_ref):
    f32 = jnp.float32
    h = (jnp.dot(x_ref[...], w1s_ref[...], preferred_element_type=f32)
         + jnp.dot(alo_ref[...], wnlo_ref[...], preferred_element_type=f32)
         + jnp.dot(ahi_ref[...], wnhi_ref[...], preferred_element_type=f32)
         + b1_ref[...])
    h = _gn_gelu(h, g1_ref[...], be1_ref[...])
    mz = jnp.dot(h, wmm_ref[...], preferred_element_type=f32) + bmm_ref[...]
    lv = jnp.dot(h, wmv_ref[...], preferred_element_type=f32) + bmv_ref[...]
    lv = jnp.clip(lv, -30.0, 20.0)
    z = mz + jnp.exp(0.5 * lv) * nz_ref[...]
    d0 = jax.nn.gelu(jnp.dot(z, wup_ref[...], preferred_element_type=f32)
                     + bup_ref[...])
    dlo_ref[...] = d0[:, :H]
    dhi_ref[...] = d0[:, H:]


def _stage_b_body(dlo_ref, dhi_ref, alo_ref, ahi_ref,
                  wslo_ref, wshi_ref, wnlo_ref, wnhi_ref,
                  b2_ref, g2_ref, be2_ref, wout_ref, bout_ref,
                  out_ref):
    f32 = jnp.float32
    d = (jnp.dot(dlo_ref[...], wslo_ref[...], preferred_element_type=f32)
         + jnp.dot(dhi_ref[...], wshi_ref[...], preferred_element_type=f32)
         + jnp.dot(alo_ref[...], wnlo_ref[...], preferred_element_type=f32)
         + jnp.dot(ahi_ref[...], wnhi_ref[...], preferred_element_type=f32)
         + b2_ref[...])
    d = _gn_gelu(d, g2_ref[...], be2_ref[...])
    out_ref[...] = (jnp.dot(d, wout_ref[...], preferred_element_type=f32)
                    + bout_ref[...])


_BR = 2000


def _row_spec(w):
    return pl.BlockSpec((_BR, w), lambda i: (i, 0))


def _full_spec(shape):
    return pl.BlockSpec(shape, lambda i: tuple(0 for _ in shape))


def _stage_a(x, alo, ahi, nz, w1s, wnlo, wnhi, b1, g1, be1,
             wmm, wmv, bmm, bmv, wup, bup):
    full = [_full_spec(a.shape) for a in
            (w1s, wnlo, wnhi, b1, g1, be1, wmm, wmv, bmm, bmv, wup, bup)]
    return pl.pallas_call(
        _stage_a_body,
        grid=(N // _BR,),
        in_specs=[_row_spec(D), _row_spec(H), _row_spec(H), _row_spec(LAT)] + full,
        out_specs=(_row_spec(H), _row_spec(H)),
        out_shape=(jax.ShapeDtypeStruct((N, H), jnp.float32),
                   jax.ShapeDtypeStruct((N, H), jnp.float32)),
    )(x, alo, ahi, nz, w1s, wnlo, wnhi, b1, g1, be1,
      wmm, wmv, bmm, bmv, wup, bup)


def _stage_b(dlo, dhi, alo, ahi, wslo, wshi, wnlo, wnhi, b2, g2, be2,
             wout, bout):
    full = [_full_spec(a.shape) for a in
            (wslo, wshi, wnlo, wnhi, b2, g2, be2, wout, bout)]
    return pl.pallas_call(
        _stage_b_body,
        grid=(N // _BR,),
        in_specs=[_row_spec(H), _row_spec(H), _row_spec(H), _row_spec(H)] + full,
        out_specs=_row_spec(OUT),
        out_shape=jax.ShapeDtypeStruct((N, OUT), jnp.float32),
    )(dlo, dhi, alo, ahi, wslo, wshi, wnlo, wnhi, b2, g2, be2, wout, bout)


def kernel(x, edge_index, W1s, W1n, b1, g1, be1, Wmu, bmu, Wup, bup,
           W2s, W2n, b2, g2, be2, Wout, bout, noise):
    ei = edge_index.astype(jnp.int32)
    pad_src = jnp.zeros((NS, PPT), jnp.int32)
    pad_dst = jnp.broadcast_to(
        N + (jnp.arange(PPT, dtype=jnp.int32) % PADROWS), (NS, PPT))
    src = jnp.concatenate([ei[0].reshape(NS, EPT), pad_src], axis=1)
    src = src.reshape(NS, CH, BK)
    dst = jnp.concatenate([ei[1].reshape(NS, EPT), pad_dst], axis=1)
    dst = dst.reshape(NS, CH, BK)
    zero_rows = jnp.zeros((SR, H), jnp.float32)

    a1_lo, a1_hi = _agg(src, dst, x[:, :H], x[:, H:], zero_rows)

    d0_lo, d0_hi = _stage_a(
        x, a1_lo, a1_hi, noise,
        W1s, W1n[:H], W1n[H:],
        b1.reshape(1, D), g1.reshape(1, D), be1.reshape(1, D),
        Wmu[:, :LAT], Wmu[:, LAT:],
        bmu[:LAT].reshape(1, LAT), bmu[LAT:].reshape(1, LAT),
        Wup, bup.reshape(1, D))

    a2_lo, a2_hi = _agg(src, dst, d0_lo, d0_hi, zero_rows)

    return _stage_b(
        d0_lo, d0_hi, a2_lo, a2_hi,
        W2s[:H], W2s[H:], W2n[:H], W2n[H:],
        b2.reshape(1, D), g2.reshape(1, D), be2.reshape(1, D),
        Wout, bout.reshape(1, OUT))
